# baseline (speedup 1.0000x reference)
import functools

import jax
import jax.numpy as jnp
from jax import lax
from jax.experimental import pallas as pl
from jax.experimental.pallas import tpu as pltpu
from jax.experimental.pallas import tpu_sc as plsc

_IN_WALL = 1
_IN_PATH = 2
_IN_SOURCE = 3
_IN_TARGET = 4
_IN_NEWLINE = 5
_OUT_IGNORE = 1
_OUT_NEWLINE = 3

_W = 64
_H = 64
_ROW = _W * (_H + 1)
_NPIX = _W * _H
_NWORKERS = 32


def _sc_body(x_hbm, y_hbm, in_hbm, lbl_hbm, xv, yv, ov_in, ov_lbl):
    spw = x_hbm.shape[0] // _NWORKERS
    wid = lax.axis_index("s") * 2 + lax.axis_index("c")
    base = wid * spw

    fives = jnp.full((16,), _IN_NEWLINE, jnp.int32)
    threes = jnp.full((16,), _OUT_NEWLINE, jnp.int32)

    def fill_body(j, c):
        ov_in[pl.ds(j * 16, 16)] = fives
        ov_lbl[pl.ds(j * 16, 16)] = threes
        return c

    lax.fori_loop(0, _ROW // 16, fill_body, 0)

    def sample_body(i, carry):
        b = base + i
        pltpu.sync_copy(x_hbm.at[b], xv)
        pltpu.sync_copy(y_hbm.at[b], yv)

        def w_body(w, c):
            ib = w * _H
            ob = w * (_H + 1)
            for hc in range(_H // 16):
                off = ib + hc * 16
                r = xv[pl.ds(off, 16)]
                g = xv[pl.ds(_NPIX + off, 16)]
                bl = xv[pl.ds(2 * _NPIX + off, 16)]
                code = (
                    jnp.where(r != 0, 4, 0)
                    + jnp.where(g != 0, 2, 0)
                    + jnp.where(bl != 0, 1, 0)
                )
                tok = jnp.where(
                    code == 0,
                    _IN_WALL,
                    jnp.where(
                        code == 7,
                        _IN_PATH,
                        jnp.where(
                            code == 4,
                            _IN_SOURCE,
                            jnp.where(code == 2, _IN_TARGET, _IN_NEWLINE),
                        ),
                    ),
                )
                ov_in[pl.ds(ob + hc * 16, 16)] = tok
                yy = yv[pl.ds(off, 16)]
                ov_lbl[pl.ds(ob + hc * 16, 16)] = yy + _OUT_IGNORE
            return c

        lax.fori_loop(0, _W, w_body, 0)
        pltpu.sync_copy(ov_in, in_hbm.at[b])
        pltpu.sync_copy(ov_lbl, lbl_hbm.at[b])
        return carry

    lax.fori_loop(0, spw, sample_body, 0)


def kernel(x, y):
    B, C, W, H = x.shape
    x2 = x.reshape(B, C * W * H)
    y2 = y.reshape(B, W * H)

    mesh = plsc.VectorSubcoreMesh(
        core_axis_name="c", subcore_axis_name="s", num_cores=2, num_subcores=16
    )
    run = pl.kernel(
        _sc_body,
        out_type=(
            jax.ShapeDtypeStruct((B, _ROW), jnp.int32),
            jax.ShapeDtypeStruct((B, _ROW), jnp.int32),
        ),
        mesh=mesh,
        scratch_types=(
            pltpu.VMEM((C * W * H,), jnp.float32),
            pltpu.VMEM((W * H,), jnp.int32),
            pltpu.VMEM((_ROW,), jnp.int32),
            pltpu.VMEM((_ROW,), jnp.int32),
        ),
    )
    input_grid, output_grid = run(x2, y2)
    return (input_grid, output_grid)


# speedup vs baseline: 1.4378x; 1.4378x over previous
import jax
import jax.numpy as jnp
from jax import lax
from jax.experimental import pallas as pl
from jax.experimental.pallas import tpu as pltpu
from jax.experimental.pallas import tpu_sc as plsc

_IN_WALL = 1
_IN_PATH = 2
_IN_SOURCE = 3
_IN_TARGET = 4
_IN_NEWLINE = 5
_OUT_IGNORE = 1
_OUT_NEWLINE = 3

_W = 64
_H = 64
_ROW = _W * (_H + 1)
_NPIX = _W * _H
_NWORKERS = 32
_NBUF = 2


def _s32(v):
    v &= 0xFFFFFFFF
    return v - (1 << 32) if v >= (1 << 31) else v


_ONE = 0x3F800000
_C_PATH = _s32((_ONE << 2) + (_ONE << 1) + _ONE)
_C_SOURCE = _s32(_ONE << 2)
_C_TARGET = _s32(_ONE << 1)


def _sc_body(
    x_hbm,
    y_hbm,
    in_hbm,
    lbl_hbm,
    xv0,
    xv1,
    yv0,
    yv1,
    ov_in0,
    ov_in1,
    ov_lbl0,
    ov_lbl1,
    sem_in,
    sem_out,
):
    xv = (xv0, xv1)
    yv = (yv0, yv1)
    ov_in = (ov_in0, ov_in1)
    ov_lbl = (ov_lbl0, ov_lbl1)
    spw = x_hbm.shape[0] // _NWORKERS
    wid = lax.axis_index("s") * 2 + lax.axis_index("c")
    base = wid * spw

    fives = jnp.full((16,), _IN_NEWLINE, jnp.int32)
    threes = jnp.full((16,), _OUT_NEWLINE, jnp.int32)
    for s in range(_NBUF):

        @pl.loop(0, _ROW // 16)
        def _(j, s=s):
            ov_in[s][pl.ds(j * 16, 16)] = fives
            ov_lbl[s][pl.ds(j * 16, 16)] = threes

    for s in range(_NBUF):
        b = base + s
        pltpu.async_copy(x_hbm.at[b], xv[s], sem_in.at[s])
        pltpu.async_copy(y_hbm.at[b], yv[s], sem_in.at[s])

    @pl.loop(0, spw, step=_NBUF)
    def _(g):
        for s in range(_NBUF):
            i = g + s
            b = base + i
            pltpu.make_async_copy(x_hbm.at[b], xv[s], sem_in.at[s]).wait()
            pltpu.make_async_copy(y_hbm.at[b], yv[s], sem_in.at[s]).wait()

            @pl.when(i >= _NBUF)
            def _(b=b, s=s):
                bp = b - _NBUF
                pltpu.make_async_copy(ov_in[s], in_hbm.at[bp], sem_out.at[s]).wait()
                pltpu.make_async_copy(ov_lbl[s], lbl_hbm.at[bp], sem_out.at[s]).wait()

            @pl.loop(0, _W)
            def _(w, s=s):
                ib = w * _H
                ob = w * (_H + 1)
                for hc in range(_H // 16):
                    off = ib + hc * 16
                    r = lax.bitcast_convert_type(xv[s][pl.ds(off, 16)], jnp.int32)
                    g_ = lax.bitcast_convert_type(
                        xv[s][pl.ds(_NPIX + off, 16)], jnp.int32
                    )
                    bl = lax.bitcast_convert_type(
                        xv[s][pl.ds(2 * _NPIX + off, 16)], jnp.int32
                    )
                    code = (r << 2) + (g_ << 1) + bl
                    tok = jnp.where(
                        code == 0,
                        _IN_WALL,
                        jnp.where(
                            code == _C_PATH,
                            _IN_PATH,
                            jnp.where(
                                code == _C_SOURCE,
                                _IN_SOURCE,
                                jnp.where(code == _C_TARGET, _IN_TARGET, _IN_NEWLINE),
                            ),
                        ),
                    )
                    ov_in[s][pl.ds(ob + hc * 16, 16)] = tok
                    ov_lbl[s][pl.ds(ob + hc * 16, 16)] = (
                        yv[s][pl.ds(off, 16)] + _OUT_IGNORE
                    )

            pltpu.async_copy(ov_in[s], in_hbm.at[b], sem_out.at[s])
            pltpu.async_copy(ov_lbl[s], lbl_hbm.at[b], sem_out.at[s])

            @pl.when(i + _NBUF < spw)
            def _(b=b, s=s):
                b2 = b + _NBUF
                pltpu.async_copy(x_hbm.at[b2], xv[s], sem_in.at[s])
                pltpu.async_copy(y_hbm.at[b2], yv[s], sem_in.at[s])

    for s in range(_NBUF):
        bl = base + spw - _NBUF + s
        pltpu.make_async_copy(ov_in[s], in_hbm.at[bl], sem_out.at[s]).wait()
        pltpu.make_async_copy(ov_lbl[s], lbl_hbm.at[bl], sem_out.at[s]).wait()


def kernel(x, y):
    B, C, W, H = x.shape
    x2 = x.reshape(B, C * W * H)
    y2 = y.reshape(B, W * H)

    mesh = plsc.VectorSubcoreMesh(
        core_axis_name="c", subcore_axis_name="s", num_cores=2, num_subcores=16
    )
    run = pl.kernel(
        _sc_body,
        out_type=(
            jax.ShapeDtypeStruct((B, _ROW), jnp.int32),
            jax.ShapeDtypeStruct((B, _ROW), jnp.int32),
        ),
        mesh=mesh,
        scratch_types=(
            pltpu.VMEM((C * W * H,), jnp.float32),
            pltpu.VMEM((C * W * H,), jnp.float32),
            pltpu.VMEM((W * H,), jnp.int32),
            pltpu.VMEM((W * H,), jnp.int32),
            pltpu.VMEM((_ROW,), jnp.int32),
            pltpu.VMEM((_ROW,), jnp.int32),
            pltpu.VMEM((_ROW,), jnp.int32),
            pltpu.VMEM((_ROW,), jnp.int32),
            pltpu.SemaphoreType.DMA((_NBUF,)),
            pltpu.SemaphoreType.DMA((_NBUF,)),
        ),
    )
    input_grid, output_grid = run(x2, y2)
    return (input_grid, output_grid)


# speedup vs baseline: 2.1208x; 1.4750x over previous
import jax
import jax.numpy as jnp
from jax import lax
from jax.experimental import pallas as pl
from jax.experimental.pallas import tpu as pltpu
from jax.experimental.pallas import tpu_sc as plsc

_IN_WALL = 1
_IN_PATH = 2
_IN_SOURCE = 3
_IN_TARGET = 4
_IN_NEWLINE = 5
_OUT_IGNORE = 1
_OUT_NEWLINE = 3

_W = 64
_H = 64
_ROW = _W * (_H + 1)
_NPIX = _W * _H
_NWORKERS = 32
_NBUF = 2


def _s32(v):
    v &= 0xFFFFFFFF
    return v - (1 << 32) if v >= (1 << 31) else v


_ONE = 0x3F800000
_C_PATH = _s32((_ONE << 2) + (_ONE << 1) + _ONE)
_C_SOURCE = _s32(_ONE << 2)
_C_TARGET = _s32(_ONE << 1)


def _sc_body(
    x_hbm,
    y_hbm,
    in_hbm,
    lbl_hbm,
    xv0,
    xv1,
    yv0,
    yv1,
    ov_in0,
    ov_in1,
    ov_lbl0,
    ov_lbl1,
    sem_in,
    sem_out,
):
    xv = (xv0, xv1)
    yv = (yv0, yv1)
    ov_in = (ov_in0, ov_in1)
    ov_lbl = (ov_lbl0, ov_lbl1)
    spw = x_hbm.shape[0] // _NWORKERS
    wid = lax.axis_index("s") * 2 + lax.axis_index("c")
    base = wid * spw

    fives = jnp.full((16,), _IN_NEWLINE, jnp.int32)
    threes = jnp.full((16,), _OUT_NEWLINE, jnp.int32)
    for s in range(_NBUF):

        @pl.loop(0, _ROW // 16)
        def _(j, s=s):
            ov_in[s][pl.ds(j * 16, 16)] = fives
            ov_lbl[s][pl.ds(j * 16, 16)] = threes

    for s in range(_NBUF):
        b = base + s
        pltpu.async_copy(x_hbm.at[b], xv[s], sem_in.at[s])
        pltpu.async_copy(y_hbm.at[b], yv[s], sem_in.at[s])

    @pl.loop(0, spw, step=_NBUF)
    def _(g):
        for s in range(_NBUF):
            i = g + s
            b = base + i
            pltpu.make_async_copy(x_hbm.at[b], xv[s], sem_in.at[s]).wait()
            pltpu.make_async_copy(y_hbm.at[b], yv[s], sem_in.at[s]).wait()

            @pl.when(i >= _NBUF)
            def _(b=b, s=s):
                bp = b - _NBUF
                pltpu.make_async_copy(ov_in[s], in_hbm.at[bp], sem_out.at[s]).wait()
                pltpu.make_async_copy(ov_lbl[s], lbl_hbm.at[bp], sem_out.at[s]).wait()

            @plsc.parallel_loop(0, _NPIX // 16, unroll=8)
            def _(t, s=s):
                off = t * 16
                ob = off + (t >> 2)
                r = lax.bitcast_convert_type(xv[s][pl.ds(off, 16)], jnp.int32)
                g_ = lax.bitcast_convert_type(
                    xv[s][pl.ds(_NPIX + off, 16)], jnp.int32
                )
                bl = lax.bitcast_convert_type(
                    xv[s][pl.ds(2 * _NPIX + off, 16)], jnp.int32
                )
                code = (r << 2) + (g_ << 1) + bl
                tok = jnp.where(
                    code == 0,
                    _IN_WALL,
                    jnp.where(
                        code == _C_PATH,
                        _IN_PATH,
                        jnp.where(
                            code == _C_SOURCE,
                            _IN_SOURCE,
                            jnp.where(code == _C_TARGET, _IN_TARGET, _IN_NEWLINE),
                        ),
                    ),
                )
                ov_in[s][pl.ds(ob, 16)] = tok
                ov_lbl[s][pl.ds(ob, 16)] = yv[s][pl.ds(off, 16)] + _OUT_IGNORE

            pltpu.async_copy(ov_in[s], in_hbm.at[b], sem_out.at[s])
            pltpu.async_copy(ov_lbl[s], lbl_hbm.at[b], sem_out.at[s])

            @pl.when(i + _NBUF < spw)
            def _(b=b, s=s):
                b2 = b + _NBUF
                pltpu.async_copy(x_hbm.at[b2], xv[s], sem_in.at[s])
                pltpu.async_copy(y_hbm.at[b2], yv[s], sem_in.at[s])

    for s in range(_NBUF):
        bl = base + spw - _NBUF + s
        pltpu.make_async_copy(ov_in[s], in_hbm.at[bl], sem_out.at[s]).wait()
        pltpu.make_async_copy(ov_lbl[s], lbl_hbm.at[bl], sem_out.at[s]).wait()


def kernel(x, y):
    B, C, W, H = x.shape
    x2 = x.reshape(B, C * W * H)
    y2 = y.reshape(B, W * H)

    mesh = plsc.VectorSubcoreMesh(
        core_axis_name="c", subcore_axis_name="s", num_cores=2, num_subcores=16
    )
    run = pl.kernel(
        _sc_body,
        out_type=(
            jax.ShapeDtypeStruct((B, _ROW), jnp.int32),
            jax.ShapeDtypeStruct((B, _ROW), jnp.int32),
        ),
        mesh=mesh,
        scratch_types=(
            pltpu.VMEM((C * W * H,), jnp.float32),
            pltpu.VMEM((C * W * H,), jnp.float32),
            pltpu.VMEM((W * H,), jnp.int32),
            pltpu.VMEM((W * H,), jnp.int32),
            pltpu.VMEM((_ROW,), jnp.int32),
            pltpu.VMEM((_ROW,), jnp.int32),
            pltpu.VMEM((_ROW,), jnp.int32),
            pltpu.VMEM((_ROW,), jnp.int32),
            pltpu.SemaphoreType.DMA((_NBUF,)),
            pltpu.SemaphoreType.DMA((_NBUF,)),
        ),
    )
    input_grid, output_grid = run(x2, y2)
    return (input_grid, output_grid)


# speedup vs baseline: 7.0500x; 3.3243x over previous
import jax
import jax.numpy as jnp
from jax import lax
from jax.experimental import pallas as pl
from jax.experimental.pallas import tpu as pltpu
from jax.experimental.pallas import tpu_sc as plsc

_IN_WALL = 1
_IN_PATH = 2
_IN_SOURCE = 3
_IN_TARGET = 4
_IN_NEWLINE = 5
_OUT_IGNORE = 1
_OUT_NEWLINE = 3

_W = 64
_H = 64
_ROW = _W * (_H + 1)
_NPIX = _W * _H
_NWORKERS = 32
_CC = 128
_GR = 8


def _s32(v):
    v &= 0xFFFFFFFF
    return v - (1 << 32) if v >= (1 << 31) else v


_ONE = 0x3F800000
_C_PATH = _s32((_ONE << 2) + (_ONE << 1) + _ONE)
_C_SOURCE = _s32(_ONE << 2)
_C_TARGET = _s32(_ONE << 1)


def _sc_body(x_hbm, out_hbm, xb0, xb1, ov, sem_in, sem_out):
    B = x_hbm.shape[1]
    ncol = B // _CC
    nunit_total = (_W // _GR) * ncol
    upw = nunit_total // _NWORKERS
    npass = upw * _GR
    xb = (xb0, xb1)
    wid = lax.axis_index("s") * 2 + lax.axis_index("c")

    def pass_gcp(q):
        u = wid * upw + (q >> 3)
        g = u >> 5
        col = (u & (ncol - 1)) * _CC
        p = q & 7
        return g, col, p

    def in_copies(q, s):
        g, col, p = pass_gcp(q)
        w = g * _GR + p
        return [
            pltpu.make_async_copy(
                x_hbm.at[pl.ds(ch * _NPIX + w * _H, _H), pl.ds(col, _CC)],
                xb[s].at[ch],
                sem_in.at[s],
            )
            for ch in range(3)
        ]

    def flush_copy(q, last):
        g, col, p = pass_gcp(q)
        ntr = 9 if last else 8
        return pltpu.make_async_copy(
            ov.at[pl.ds(p * 8, ntr)],
            out_hbm.at[pl.ds(g * 65 + p * 8, ntr), :, pl.ds(col, _CC)],
            sem_out,
        )

    def issue_flush(q):
        g, col, p = pass_gcp(q)

        @pl.when(p < 7)
        def _():
            flush_copy(q, False).start()

        @pl.when(p == 7)
        def _():
            flush_copy(q, True).start()

    def wait_flush(q):
        g, col, p = pass_gcp(q)

        @pl.when(p < 7)
        def _():
            flush_copy(q, False).wait()

        @pl.when(p == 7)
        def _():
            flush_copy(q, True).wait()

    fives = jnp.full((16,), _IN_NEWLINE, jnp.int32)
    for p in range(_GR):
        lj = p * 65 + _H
        for l in range(_CC // 16):
            ov[lj >> 3, lj & 7, pl.ds(l * 16, 16)] = fives

    for s in range(2):
        for c in in_copies(s, s):
            c.start()

    @pl.loop(0, npass, step=2)
    def _(q0):
        for s in range(2):
            q = q0 + s
            for c in in_copies(q, s):
                c.wait()

            @pl.when(q >= _GR)
            def _(q=q):
                wait_flush(q - _GR)

            g, col, p = pass_gcp(q)
            base = p * 65

            @plsc.parallel_loop(0, _H * _CC // 16, unroll=8)
            def _(t, s=s, base=base):
                h = t >> 3
                off = (t & 7) * 16
                lj = base + h
                r = lax.bitcast_convert_type(xb[s][0, h, pl.ds(off, 16)], jnp.int32)
                g_ = lax.bitcast_convert_type(xb[s][1, h, pl.ds(off, 16)], jnp.int32)
                bl = lax.bitcast_convert_type(xb[s][2, h, pl.ds(off, 16)], jnp.int32)
                code = (r << 2) + (g_ << 1) + bl
                tok = jnp.where(
                    code == 0,
                    _IN_WALL,
                    jnp.where(
                        code == _C_PATH,
                        _IN_PATH,
                        jnp.where(
                            code == _C_SOURCE,
                            _IN_SOURCE,
                            jnp.where(code == _C_TARGET, _IN_TARGET, _IN_NEWLINE),
                        ),
                    ),
                )
                ov[lj >> 3, lj & 7, pl.ds(off, 16)] = tok

            issue_flush(q)

            @pl.when(q + 2 < npass)
            def _(q=q, s=s):
                for c in in_copies(q + 2, s):
                    c.start()

    for p in range(_GR):
        flush_copy(npass - _GR + p, p == 7).wait()


def _tc_lbl_body(y_ref, o_ref):
    for wl in range(8):
        o_ref[pl.ds(wl * (_H + 1), _H), :] = y_ref[pl.ds(wl * _H, _H), :] + _OUT_IGNORE
        o_ref[pl.ds(wl * (_H + 1) + _H, 1), :] = jnp.full(
            (1, o_ref.shape[1]), _OUT_NEWLINE, jnp.int32
        )


def kernel(x, y):
    B, C, W, H = x.shape
    xt = x.transpose(1, 2, 3, 0).reshape(C * W * H, B)
    yt = y.transpose(1, 2, 0).reshape(W * H, B)

    mesh = plsc.VectorSubcoreMesh(
        core_axis_name="c", subcore_axis_name="s", num_cores=2, num_subcores=16
    )
    sc_run = pl.kernel(
        _sc_body,
        out_type=jax.ShapeDtypeStruct((_ROW // 8, 8, B), jnp.int32),
        mesh=mesh,
        scratch_types=(
            pltpu.VMEM((3, _H, _CC), jnp.float32),
            pltpu.VMEM((3, _H, _CC), jnp.float32),
            pltpu.VMEM((65, 8, _CC), jnp.int32),
            pltpu.SemaphoreType.DMA((2,)),
            pltpu.SemaphoreType.DMA,
        ),
    )
    in_t = sc_run(xt)

    cb = 1024
    lbl_t = pl.pallas_call(
        _tc_lbl_body,
        out_shape=jax.ShapeDtypeStruct((_ROW, B), jnp.int32),
        grid=(8, B // cb),
        in_specs=[pl.BlockSpec((8 * _H, cb), lambda i, j: (i, j))],
        out_specs=pl.BlockSpec((8 * (_H + 1), cb), lambda i, j: (i, j)),
    )(yt)

    input_grid = in_t.reshape(_ROW, B).T
    output_grid = lbl_t.T
    return (input_grid, output_grid)


# speedup vs baseline: 7.0673x; 1.0025x over previous
import jax
import jax.numpy as jnp
from jax import lax
from jax.experimental import pallas as pl
from jax.experimental.pallas import tpu as pltpu
from jax.experimental.pallas import tpu_sc as plsc

_IN_WALL = 1
_IN_PATH = 2
_IN_SOURCE = 3
_IN_TARGET = 4
_IN_NEWLINE = 5
_OUT_IGNORE = 1
_OUT_NEWLINE = 3

_W = 64
_H = 64
_ROW = _W * (_H + 1)
_NPIX = _W * _H
_NWORKERS = 32
_CC = 128
_GR = 8


def _s32(v):
    v &= 0xFFFFFFFF
    return v - (1 << 32) if v >= (1 << 31) else v


_ONE = 0x3F800000
_C_PATH = _s32((_ONE << 2) + (_ONE << 1) + _ONE)
_C_SOURCE = _s32(_ONE << 2)
_C_TARGET = _s32(_ONE << 1)


def _sc_body(x_hbm, out_hbm, xb0, xb1, ov, sem_in, sem_out):
    B = x_hbm.shape[1]
    ncol = B // _CC
    nunit_total = (_W // _GR) * ncol
    upw = nunit_total // _NWORKERS
    npass = upw * _GR
    xb = (xb0, xb1)
    wid = lax.axis_index("s") * 2 + lax.axis_index("c")

    def pass_gcp(q):
        u = wid * upw + (q >> 3)
        g = u >> 5
        col = (u & (ncol - 1)) * _CC
        p = q & 7
        return g, col, p

    def in_copies(q, s):
        g, col, p = pass_gcp(q)
        w = g * _GR + p
        return [
            pltpu.make_async_copy(
                x_hbm.at[pl.ds(ch * _NPIX + w * _H, _H), pl.ds(col, _CC)],
                xb[s].at[ch],
                sem_in.at[s],
            )
            for ch in range(3)
        ]

    def flush_copy(q, last):
        g, col, p = pass_gcp(q)
        ntr = 9 if last else 8
        return pltpu.make_async_copy(
            ov.at[pl.ds(p * 8, ntr)],
            out_hbm.at[pl.ds(g * 65 + p * 8, ntr), :, pl.ds(col, _CC)],
            sem_out,
        )

    def issue_flush(q):
        g, col, p = pass_gcp(q)

        @pl.when(p < 7)
        def _():
            flush_copy(q, False).start()

        @pl.when(p == 7)
        def _():
            flush_copy(q, True).start()

    def wait_flush(q):
        g, col, p = pass_gcp(q)

        @pl.when(p < 7)
        def _():
            flush_copy(q, False).wait()

        @pl.when(p == 7)
        def _():
            flush_copy(q, True).wait()

    lane = lax.iota(jnp.int32, 16)
    lut = jnp.where(
        lane == 0,
        _IN_WALL,
        jnp.where(lane == 7, _IN_PATH, jnp.minimum(lane + 2, _IN_NEWLINE)),
    )

    fives = jnp.full((16,), _IN_NEWLINE, jnp.int32)
    for p in range(_GR):
        lj = p * 65 + _H
        for l in range(_CC // 16):
            ov[lj >> 3, lj & 7, pl.ds(l * 16, 16)] = fives

    for s in range(2):
        for c in in_copies(s, s):
            c.start()

    @pl.loop(0, npass, step=2)
    def _(q0):
        for s in range(2):
            q = q0 + s
            for c in in_copies(q, s):
                c.wait()

            @pl.when(q >= _GR)
            def _(q=q):
                wait_flush(q - _GR)

            g, col, p = pass_gcp(q)
            base = p * 65

            @plsc.parallel_loop(0, _H * _CC // 16, unroll=16)
            def _(t, s=s, base=base):
                h = t >> 3
                off = (t & 7) * 16
                lj = base + h
                r = xb[s][0, h, pl.ds(off, 16)]
                g_ = xb[s][1, h, pl.ds(off, 16)]
                bl = xb[s][2, h, pl.ds(off, 16)]
                code = (r + (g_ + g_) + (bl + bl) + (bl + bl)).astype(jnp.int32)
                tok = lax.gather(
                    lut,
                    code[:, None],
                    dimension_numbers=lax.GatherDimensionNumbers(
                        offset_dims=(),
                        collapsed_slice_dims=(0,),
                        start_index_map=(0,),
                    ),
                    slice_sizes=(1,),
                    mode=lax.GatherScatterMode.PROMISE_IN_BOUNDS,
                )
                ov[lj >> 3, lj & 7, pl.ds(off, 16)] = tok

            issue_flush(q)

            @pl.when(q + 2 < npass)
            def _(q=q, s=s):
                for c in in_copies(q + 2, s):
                    c.start()

    for p in range(_GR):
        flush_copy(npass - _GR + p, p == 7).wait()


def _tc_lbl_body(y_ref, o_ref):
    for wl in range(8):
        o_ref[pl.ds(wl * (_H + 1), _H), :] = y_ref[pl.ds(wl * _H, _H), :] + _OUT_IGNORE
        o_ref[pl.ds(wl * (_H + 1) + _H, 1), :] = jnp.full(
            (1, o_ref.shape[1]), _OUT_NEWLINE, jnp.int32
        )


def kernel(x, y):
    B, C, W, H = x.shape
    xt = x.transpose(1, 2, 3, 0).reshape(C * W * H, B)
    yt = y.transpose(1, 2, 0).reshape(W * H, B)

    mesh = plsc.VectorSubcoreMesh(
        core_axis_name="c", subcore_axis_name="s", num_cores=2, num_subcores=16
    )
    sc_run = pl.kernel(
        _sc_body,
        out_type=jax.ShapeDtypeStruct((_ROW // 8, 8, B), jnp.int32),
        mesh=mesh,
        scratch_types=(
            pltpu.VMEM((3, _H, _CC), jnp.float32),
            pltpu.VMEM((3, _H, _CC), jnp.float32),
            pltpu.VMEM((65, 8, _CC), jnp.int32),
            pltpu.SemaphoreType.DMA((2,)),
            pltpu.SemaphoreType.DMA,
        ),
    )
    in_t = sc_run(xt)

    cb = 1024
    lbl_t = pl.pallas_call(
        _tc_lbl_body,
        out_shape=jax.ShapeDtypeStruct((_ROW, B), jnp.int32),
        grid=(8, B // cb),
        in_specs=[pl.BlockSpec((8 * _H, cb), lambda i, j: (i, j))],
        out_specs=pl.BlockSpec((8 * (_H + 1), cb), lambda i, j: (i, j)),
    )(yt)

    input_grid = in_t.reshape(_ROW, B).T
    output_grid = lbl_t.T
    return (input_grid, output_grid)
